# Initial kernel scaffold; baseline (speedup 1.0000x reference)
import jax
import jax.numpy as jnp
from jax.experimental import pallas as pl


def kernel(atom_in_fea, nbr_fea, nbr_fea_idx, W_fc, b_fc, gamma1, beta1, gamma2, beta2):
    raise NotImplementedError("write your pallas kernel here")


# speedup vs baseline: 2.3402x; 2.3402x over previous
import functools

import jax
import jax.numpy as jnp
from jax import lax
from jax.experimental import pallas as pl
from jax.experimental.pallas import tpu as pltpu
from jax.experimental.pallas import tpu_sc as plsc

N = 10000
M = 32
D = 128
DE = 16
F = 256
NM = N * M

NC = 2
NS = 16
NW = NC * NS
CHUNK = 128
NCHUNKS = NM // CHUNK

B = 200
BM = B * M
NB = N // B

@functools.lru_cache(maxsize=1)
def _make_sc_gather():
    mesh = plsc.VectorSubcoreMesh(core_axis_name="c", subcore_axis_name="s",
                                  num_cores=NC, num_subcores=NS)

    @functools.partial(
        pl.kernel,
        out_type=jax.ShapeDtypeStruct((NM, D), jnp.float32),
        mesh=mesh,
        scratch_types=[
            pltpu.VMEM((CHUNK,), jnp.int32),
            pltpu.VMEM((CHUNK, D), jnp.float32),
            pltpu.SemaphoreType.DMA,
        ],
    )
    def sc_gather(table_hbm, idx_hbm, out_hbm, idx_v, rows_v, sem):
        wid = lax.axis_index("s") * NC + lax.axis_index("c")
        iters = (NCHUNKS + NW - 1) // NW

        def body(i, _):
            k = wid + i * NW

            @pl.when(k < NCHUNKS)
            def _do():
                pltpu.sync_copy(idx_hbm.at[k], idx_v)
                pltpu.async_copy(table_hbm.at[idx_v], rows_v, sem).wait()
                pltpu.sync_copy(rows_v, out_hbm.at[pl.ds(k * CHUNK, CHUNK)])

            return 0

        lax.fori_loop(0, iters, body, 0)

    return sc_gather


def _stats_body(atom_ref, g_ref, nbrT_ref, idxf_ref, ws_ref, wn_ref, we_ref,
                b_ref, base_ref, ssum_ref, ssq_ref):
    pid = pl.program_id(0)
    atom = atom_ref[...]
    base = jnp.dot(atom, ws_ref[...], preferred_element_type=jnp.float32)
    base = base + b_ref[...]
    base_ref[...] = base

    maskT = (idxf_ref[...] != 0).astype(jnp.float32)
    nbrT = nbrT_ref[...] * maskT
    x = jnp.dot(g_ref[...], wn_ref[...], preferred_element_type=jnp.float32)
    x = x + lax.dot_general(nbrT, we_ref[...],
                            (((0,), (0,)), ((), ())),
                            preferred_element_type=jnp.float32)
    rep = jnp.broadcast_to(base[:, None, :], (B, M, F)).reshape(BM, F)
    tg = rep + x

    @pl.when(pid == 0)
    def _init():
        ssum_ref[...] = jnp.zeros_like(ssum_ref)
        ssq_ref[...] = jnp.zeros_like(ssq_ref)

    ssum_ref[...] += jnp.sum(tg, axis=0, keepdims=True)
    ssq_ref[...] += jnp.sum(tg * tg, axis=0, keepdims=True)


def _main_body(base_ref, g_ref, nbrT_ref, idxf_ref, idx_ref, wn_ref, we_ref,
               a_ref, bb_ref, s_ref, tsum_ref, tsq_ref):
    pid = pl.program_id(0)
    maskT = (idxf_ref[...] != 0).astype(jnp.float32)
    nbrT = nbrT_ref[...] * maskT
    x = jnp.dot(g_ref[...], wn_ref[...], preferred_element_type=jnp.float32)
    x = x + lax.dot_general(nbrT, we_ref[...],
                            (((0,), (0,)), ((), ())),
                            preferred_element_type=jnp.float32)
    yb = base_ref[...] * a_ref[...] + bb_ref[...]
    rep = jnp.broadcast_to(yb[:, None, :], (B, M, F)).reshape(BM, F)
    y = rep + x

    p = jax.nn.sigmoid(y[:, :D]) * jax.nn.softplus(y[:, D:])
    psum = jnp.sum(p.reshape(B, M, D), axis=1)
    cnt0 = jnp.sum((idx_ref[...] == 0).astype(jnp.float32), axis=1,
                   keepdims=True)
    corr = jax.nn.sigmoid(yb[:, :D]) * jax.nn.softplus(yb[:, D:])
    s = psum - cnt0 * corr
    s_ref[...] = s

    @pl.when(pid == 0)
    def _init():
        tsum_ref[...] = jnp.zeros_like(tsum_ref)
        tsq_ref[...] = jnp.zeros_like(tsq_ref)

    tsum_ref[...] += jnp.sum(s, axis=0, keepdims=True)
    tsq_ref[...] += jnp.sum(s * s, axis=0, keepdims=True)


def _final_body(atom_ref, s_ref, a2_ref, bb2_ref, out_ref):
    y2 = s_ref[...] * a2_ref[...] + bb2_ref[...]
    out_ref[...] = jax.nn.softplus(atom_ref[...] + y2)


def kernel(atom_in_fea, nbr_fea, nbr_fea_idx, W_fc, b_fc, gamma1, beta1,
           gamma2, beta2):
    idx = nbr_fea_idx.astype(jnp.int32)
    iflat = jnp.where(idx == 0, N, idx).reshape(NCHUNKS, CHUNK)
    table = jnp.concatenate(
        [atom_in_fea, jnp.zeros((1, D), jnp.float32)], axis=0)
    nbrT = jnp.transpose(nbr_fea, (2, 0, 1)).reshape(DE, NM)
    idxf = idx.reshape(1, NM)

    Ws = W_fc[:, :D].T
    Wn = W_fc[:, D:2 * D].T
    We = W_fc[:, 2 * D:].T
    bvec = b_fc.reshape(1, F)

    g = _make_sc_gather()(table, iflat)

    base, ssum, ssq = pl.pallas_call(
        _stats_body,
        grid=(NB,),
        in_specs=[
            pl.BlockSpec((B, D), lambda b: (b, 0)),
            pl.BlockSpec((BM, D), lambda b: (b, 0)),
            pl.BlockSpec((DE, BM), lambda b: (0, b)),
            pl.BlockSpec((1, BM), lambda b: (0, b)),
            pl.BlockSpec((D, F), lambda b: (0, 0)),
            pl.BlockSpec((D, F), lambda b: (0, 0)),
            pl.BlockSpec((DE, F), lambda b: (0, 0)),
            pl.BlockSpec((1, F), lambda b: (0, 0)),
        ],
        out_specs=[
            pl.BlockSpec((B, F), lambda b: (b, 0)),
            pl.BlockSpec((1, F), lambda b: (0, 0)),
            pl.BlockSpec((1, F), lambda b: (0, 0)),
        ],
        out_shape=[
            jax.ShapeDtypeStruct((N, F), jnp.float32),
            jax.ShapeDtypeStruct((1, F), jnp.float32),
            jax.ShapeDtypeStruct((1, F), jnp.float32),
        ],
    )(atom_in_fea, g, nbrT, idxf, Ws, Wn, We, bvec)

    mu1 = ssum / NM
    var1 = ssq / NM - mu1 * mu1
    a1 = lax.rsqrt(var1 + 1e-5) * gamma1.reshape(1, F)
    bb1 = beta1.reshape(1, F) - mu1 * a1
    Wn_s = Wn * a1
    We_s = We * a1

    s, tsum, tsq = pl.pallas_call(
        _main_body,
        grid=(NB,),
        in_specs=[
            pl.BlockSpec((B, F), lambda b: (b, 0)),
            pl.BlockSpec((BM, D), lambda b: (b, 0)),
            pl.BlockSpec((DE, BM), lambda b: (0, b)),
            pl.BlockSpec((1, BM), lambda b: (0, b)),
            pl.BlockSpec((B, M), lambda b: (b, 0)),
            pl.BlockSpec((D, F), lambda b: (0, 0)),
            pl.BlockSpec((DE, F), lambda b: (0, 0)),
            pl.BlockSpec((1, F), lambda b: (0, 0)),
            pl.BlockSpec((1, F), lambda b: (0, 0)),
        ],
        out_specs=[
            pl.BlockSpec((B, D), lambda b: (b, 0)),
            pl.BlockSpec((1, D), lambda b: (0, 0)),
            pl.BlockSpec((1, D), lambda b: (0, 0)),
        ],
        out_shape=[
            jax.ShapeDtypeStruct((N, D), jnp.float32),
            jax.ShapeDtypeStruct((1, D), jnp.float32),
            jax.ShapeDtypeStruct((1, D), jnp.float32),
        ],
    )(base, g, nbrT, idxf, idx, Wn_s, We_s, a1, bb1)

    mu2 = tsum / N
    var2 = tsq / N - mu2 * mu2
    a2 = lax.rsqrt(var2 + 1e-5) * gamma2.reshape(1, D)
    bb2 = beta2.reshape(1, D) - mu2 * a2

    out = pl.pallas_call(
        _final_body,
        grid=(NB,),
        in_specs=[
            pl.BlockSpec((B, D), lambda b: (b, 0)),
            pl.BlockSpec((B, D), lambda b: (b, 0)),
            pl.BlockSpec((1, D), lambda b: (0, 0)),
            pl.BlockSpec((1, D), lambda b: (0, 0)),
        ],
        out_specs=pl.BlockSpec((B, D), lambda b: (b, 0)),
        out_shape=jax.ShapeDtypeStruct((N, D), jnp.float32),
    )(atom_in_fea, s, a2, bb2)
    return out
